# baseline (speedup 1.0000x reference)
import functools

import jax
import jax.numpy as jnp
from jax.experimental import pallas as pl

HID = 128
N_ATOM = 10000
N_QUERY = 50000


def _affine_silu_body(x_ref, w_ref, b_ref, o_ref):
    o_ref[...] = jax.nn.silu(
        jnp.dot(x_ref[...], w_ref[...], preferred_element_type=jnp.float32)
        + b_ref[...])


def _affine_silu(x, w, b, block=1024):
    n, k = x.shape
    m = w.shape[1]
    return pl.pallas_call(
        _affine_silu_body,
        grid=(pl.cdiv(n, block),),
        in_specs=[
            pl.BlockSpec((block, k), lambda i: (i, 0)),
            pl.BlockSpec((k, m), lambda i: (0, 0)),
            pl.BlockSpec((1, m), lambda i: (0, 0)),
        ],
        out_specs=pl.BlockSpec((block, m), lambda i: (i, 0)),
        out_shape=jax.ShapeDtypeStruct((n, m), jnp.float32),
    )(x, w, b.reshape(1, -1))


def _update_body(h_ref, agg_ref, wa_ref, wb_ref, b1_ref, w2_ref, b2_ref, o_ref):
    h = h_ref[...]
    u = jax.nn.silu(
        jnp.dot(h, wa_ref[...], preferred_element_type=jnp.float32)
        + jnp.dot(agg_ref[...], wb_ref[...], preferred_element_type=jnp.float32)
        + b1_ref[...])
    o_ref[...] = h + jnp.dot(u, w2_ref[...], preferred_element_type=jnp.float32) + b2_ref[...]


def _update(h, agg, p, block=1024):
    n = h.shape[0]
    wa = p['Wu1'][:HID]
    wb = p['Wu1'][HID:]
    return pl.pallas_call(
        _update_body,
        grid=(pl.cdiv(n, block),),
        in_specs=[
            pl.BlockSpec((block, HID), lambda i: (i, 0)),
            pl.BlockSpec((block, HID), lambda i: (i, 0)),
            pl.BlockSpec((HID, HID), lambda i: (0, 0)),
            pl.BlockSpec((HID, HID), lambda i: (0, 0)),
            pl.BlockSpec((1, HID), lambda i: (0, 0)),
            pl.BlockSpec((HID, HID), lambda i: (0, 0)),
            pl.BlockSpec((1, HID), lambda i: (0, 0)),
        ],
        out_specs=pl.BlockSpec((block, HID), lambda i: (i, 0)),
        out_shape=jax.ShapeDtypeStruct((n, HID), jnp.float32),
    )(h, agg, wa, wb, p['bu1'].reshape(1, -1), p['Wu2'], p['bu2'].reshape(1, -1))


def _head_body(h_ref, w1_ref, b1_ref, w2_ref, b2_ref, o_ref):
    t = jax.nn.silu(
        jnp.dot(h_ref[...], w1_ref[...], preferred_element_type=jnp.float32)
        + b1_ref[...])
    o_ref[...] = jnp.dot(t, w2_ref[...], preferred_element_type=jnp.float32) + b2_ref[...]


def _head(h, params, block=1024):
    n = h.shape[0]
    hh = HID // 2
    return pl.pallas_call(
        _head_body,
        grid=(pl.cdiv(n, block),),
        in_specs=[
            pl.BlockSpec((block, HID), lambda i: (i, 0)),
            pl.BlockSpec((HID, hh), lambda i: (0, 0)),
            pl.BlockSpec((1, hh), lambda i: (0, 0)),
            pl.BlockSpec((hh, 1), lambda i: (0, 0)),
            pl.BlockSpec((1, 1), lambda i: (0, 0)),
        ],
        out_specs=pl.BlockSpec((block, 1), lambda i: (i, 0)),
        out_shape=jax.ShapeDtypeStruct((n, 1), jnp.float32),
    )(h, params['Wh1'], params['bh1'].reshape(1, -1),
      params['Wh2'], params['bh2'].reshape(1, -1))


def _round(p, h_src, h_dst, src, dst, e, n_dst):
    hm = _affine_silu(h_src, p['Wm'], p['bm'])
    m = jnp.take(hm, src, axis=0) * e
    agg = jax.ops.segment_sum(m, dst, num_segments=n_dst)
    return _update(h_dst, agg, p)


def kernel(z, bond_edge_index, bond_edge_attr, aq_edge_index, aq_edge_attr,
           qq_edge_index, qq_edge_attr, n_query, params):
    p = params
    h_atom = jnp.take(p['emb'], z, axis=0)

    e_bond = _affine_silu(bond_edge_attr, p['bond']['Wr'], p['bond']['br'])
    for _ in range(2):
        h_atom = _round(p['bond'], h_atom, h_atom,
                        bond_edge_index[0], bond_edge_index[1], e_bond, N_ATOM)

    e_aq = _affine_silu(aq_edge_attr, p['aq']['Wr'], p['aq']['br'])
    h_query = jnp.zeros((N_QUERY, HID), jnp.float32)
    for _ in range(3):
        h_query = _round(p['aq'], h_atom, h_query,
                         aq_edge_index[0], aq_edge_index[1], e_aq, N_QUERY)

    e_qq = _affine_silu(qq_edge_attr, p['qq']['Wr'], p['qq']['br'])
    for _ in range(2):
        h_query = _round(p['qq'], h_query, h_query,
                         qq_edge_index[0], qq_edge_index[1], e_qq, N_QUERY)

    return _head(h_query, p).reshape(N_QUERY)
